# Initial kernel scaffold; baseline (speedup 1.0000x reference)
import jax
import jax.numpy as jnp
from jax.experimental import pallas as pl


def kernel(x, theta, bn_gamma, bn_beta):
    raise NotImplementedError("write your pallas kernel here")


# speedup vs baseline: 117.9308x; 117.9308x over previous
import functools

import jax
import jax.numpy as jnp
from jax import lax
from jax.experimental import pallas as pl

B, C, L = 4, 768, 1024
OUT = 159
EPS = 1e-5


def _graph_body(x_ref, theta_ref, out_ref):
    xi = x_ref[0]
    theta = theta_ref[...]

    sq = jnp.sum(xi * xi, axis=1, keepdims=True)
    g = lax.dot_general(xi, xi, (((1,), (1,)), ((), ())),
                        preferred_element_type=jnp.float32)
    d = sq - 2.0 * g + sq.T

    col = lax.broadcasted_iota(jnp.int32, (L, L), 1)
    m1 = jnp.min(d, axis=1, keepdims=True)
    a1 = jnp.min(jnp.where(d == m1, col, L), axis=1)
    d2 = jnp.where(col == a1[:, None], jnp.inf, d)
    m2 = jnp.min(d2, axis=1, keepdims=True)
    a2 = jnp.min(jnp.where(d2 == m2, col, L), axis=1)

    e_idx = lax.iota(jnp.int32, L)
    mself = jnp.logical_and(a1 != e_idx, a2 != e_idx)
    a = ((col == a1[:, None]) | (col == a2[:, None])
         | ((col == e_idx[:, None]) & mself[:, None])).astype(jnp.float32)

    xt = jnp.dot(xi, theta, preferred_element_type=jnp.float32)

    colcnt = 2.0 + mself.astype(jnp.float32)
    xe = jnp.dot(a, xt, preferred_element_type=jnp.float32) / colcnt[:, None]
    deg = jnp.sum(a, axis=0)
    xn = lax.dot_general(a, xe, (((0,), (0,)), ((), ())),
                         preferred_element_type=jnp.float32) / deg[:, None]
    out_ref[0] = xn


def _bn_body(z_ref, gamma_ref, beta_ref, out_ref):
    z = z_ref[...]
    mean = jnp.mean(z, axis=(0, 2), keepdims=True)
    var = jnp.mean((z - mean) ** 2, axis=(0, 2), keepdims=True)
    y = (z - mean) * lax.rsqrt(var + EPS)
    y = y * gamma_ref[...][None, :, None] + beta_ref[...][None, :, None]
    out_ref[...] = jnp.maximum(y, 0.0)


@jax.jit
def kernel(x, theta, bn_gamma, bn_beta):
    xr = x.reshape(B, L, C)
    xn = pl.pallas_call(
        _graph_body,
        grid=(B,),
        in_specs=[
            pl.BlockSpec((1, L, C), lambda i: (i, 0, 0)),
            pl.BlockSpec((C, OUT), lambda i: (0, 0)),
        ],
        out_specs=pl.BlockSpec((1, L, OUT), lambda i: (i, 0, 0)),
        out_shape=jax.ShapeDtypeStruct((B, L, OUT), jnp.float32),
    )(xr, theta)

    z = xn.reshape(B, OUT, L)
    y = pl.pallas_call(
        _bn_body,
        out_shape=jax.ShapeDtypeStruct((B, OUT, L), jnp.float32),
    )(z, bn_gamma, bn_beta)
    return y[..., None]
